# baseline (speedup 1.0000x reference)
import functools

import jax
import jax.numpy as jnp
from jax import lax
from jax.experimental import pallas as pl
from jax.experimental.pallas import tpu as pltpu
from jax.experimental.pallas import tpu_sc as plsc

COMMIT = 0.25


def _encode_body(z_ref, cb_ref, idx_ref, loss_ref, cnorm_ref):
    b = pl.program_id(0)
    zb = z_ref[...]
    cb = cb_ref[...]
    T, D = zb.shape
    K = cb.shape[0]
    LW = 128

    @pl.when(b == 0)
    def _():
        cnorm_ref[...] = jnp.sum(cb * cb, axis=1)[None, :]

    mm = lax.dot_general(zb, cb, (((1,), (1,)), ((), ())),
                         preferred_element_type=jnp.float32)
    znorm = jnp.sum(zb * zb, axis=1, keepdims=True)
    TT = 64
    lane = lax.broadcasted_iota(jnp.int32, (TT, LW), 1)
    losspart = jnp.zeros((1, 1), jnp.float32)
    for t in range(0, T, TT):
        zn = znorm[t:t + TT, :]
        m = argk = None
        for j in range(K // LW):
            cn = cnorm_ref[:, j * LW:(j + 1) * LW]
            dj = zn - 2.0 * mm[t:t + TT, j * LW:(j + 1) * LW] + cn
            if j == 0:
                m, argk = dj, lane
            else:
                lt = dj < m
                m = jnp.where(lt, dj, m)
                argk = jnp.where(lt, lane + j * LW, argk)
        mind = jnp.min(m, axis=1, keepdims=True)
        idx = jnp.min(jnp.where(m == mind, argk, K), axis=1, keepdims=True)
        idx_ref[0, t:t + TT, :] = idx
        losspart += jnp.sum(mind).reshape(1, 1)
    mind = losspart

    @pl.when(b == 0)
    def _():
        loss_ref[...] = jnp.zeros((1, 1), jnp.float32)

    loss_ref[...] += jnp.sum(mind).reshape(1, 1)


_TB = 2304


def _encode(zf, codebook):
    N, D = zf.shape
    K = codebook.shape[0]
    G = N // _TB
    idx3, losssum = pl.pallas_call(
        _encode_body,
        grid=(G,),
        in_specs=[
            pl.BlockSpec((_TB, D), lambda i: (i, 0)),
            pl.BlockSpec((K, D), lambda i: (0, 0)),
        ],
        out_specs=[
            pl.BlockSpec((1, _TB, 1), lambda i: (i, 0, 0)),
            pl.BlockSpec((1, 1), lambda i: (0, 0)),
        ],
        out_shape=[
            jax.ShapeDtypeStruct((G, _TB, 1), jnp.int32),
            jax.ShapeDtypeStruct((1, 1), jnp.float32),
        ],
        scratch_shapes=[pltpu.VMEM((1, K), jnp.float32)],
    )(zf, codebook)
    return idx3.reshape(N), losssum[0, 0]


_NC, _NS = 2, 16
_NW = _NC * _NS
_CHUNK = 96


def _sc_gather_body(base, per, cb_hbm, idx_hbm, out_hbm, idx_v, buf0, buf1,
                    sem0, sem1):
    wid = lax.axis_index("s") * _NC + lax.axis_index("c")
    wbase = wid * per
    pltpu.sync_copy(idx_hbm.at[pl.ds(wbase, per)], idx_v)
    bufs = (buf0, buf1)
    sems = (sem0, sem1)
    nch = per // _CHUNK
    copies = [None] * nch
    copies[0] = pltpu.async_copy(cb_hbm.at[idx_v.at[pl.ds(0, _CHUNK)]],
                                 bufs[0], sems[0])
    for j in range(nch):
        if j + 1 < nch:
            copies[j + 1] = pltpu.async_copy(
                cb_hbm.at[idx_v.at[pl.ds((j + 1) * _CHUNK, _CHUNK)]],
                bufs[(j + 1) % 2], sems[(j + 1) % 2])
        copies[j].wait()
        pltpu.sync_copy(bufs[j % 2],
                        out_hbm.at[pl.ds(base + wbase + j * _CHUNK, _CHUNK)])


def _sc_gather_into(qref, codebook, idx_slice, base):
    K, D = codebook.shape
    per = idx_slice.shape[0] // _NW
    mesh = plsc.VectorSubcoreMesh(core_axis_name="c", subcore_axis_name="s")
    fn = functools.partial(
        pl.kernel,
        out_type=(),
        mesh=mesh,
        scratch_types=[
            pltpu.VMEM((per,), jnp.int32),
            pltpu.VMEM((_CHUNK, D), jnp.float32),
            pltpu.VMEM((_CHUNK, D), jnp.float32),
            pltpu.SemaphoreType.DMA,
            pltpu.SemaphoreType.DMA,
        ],
    )(functools.partial(_sc_gather_body, base, per))
    fn(codebook, idx_slice, qref)


_S = 2


def kernel(z, codebook):
    B, T, D = z.shape
    zf = z.reshape(-1, D)
    N = zf.shape[0]
    half = N // _S
    qref = jax.new_ref(jnp.zeros((N, D), jnp.float32))
    idx_parts, loss_parts = [], []
    for s in range(_S):
        idx_s, ls = _encode(zf[s * half:(s + 1) * half], codebook)
        _sc_gather_into(qref, codebook, idx_s, s * half)
        idx_parts.append(idx_s)
        loss_parts.append(ls)
    losssum = loss_parts[0]
    for ls in loss_parts[1:]:
        losssum = losssum + ls
    loss = losssum * ((1.0 + COMMIT) / (B * T * D))
    idx = jnp.concatenate(idx_parts).reshape(B, T)
    return qref[...].reshape(B, T, D), loss, idx


# speedup vs baseline: 1.3102x; 1.3102x over previous
import functools

import jax
import jax.numpy as jnp
from jax import lax
from jax.experimental import pallas as pl
from jax.experimental.pallas import tpu as pltpu
from jax.experimental.pallas import tpu_sc as plsc

COMMIT = 0.25


def _encode_body(z_ref, cb_ref, idx_ref, loss_ref, cnorm_ref):
    b = pl.program_id(0)
    zb = z_ref[...]
    cb = cb_ref[...]
    T, D = zb.shape
    K = cb.shape[0]
    LW = 128

    @pl.when(b == 0)
    def _():
        cnorm_ref[...] = jnp.sum(cb * cb, axis=1)[None, :]

    mm = lax.dot_general(zb, cb, (((1,), (1,)), ((), ())),
                         preferred_element_type=jnp.float32)
    znorm = jnp.sum(zb * zb, axis=1, keepdims=True)
    TT = 64
    lane = lax.broadcasted_iota(jnp.int32, (TT, LW), 1)
    losspart = jnp.zeros((1, 1), jnp.float32)
    for t in range(0, T, TT):
        zn = znorm[t:t + TT, :]
        m = argk = None
        for j in range(K // LW):
            cn = cnorm_ref[:, j * LW:(j + 1) * LW]
            dj = zn - 2.0 * mm[t:t + TT, j * LW:(j + 1) * LW] + cn
            if j == 0:
                m, argk = dj, lane
            else:
                lt = dj < m
                m = jnp.where(lt, dj, m)
                argk = jnp.where(lt, lane + j * LW, argk)
        mind = jnp.min(m, axis=1, keepdims=True)
        idx = jnp.min(jnp.where(m == mind, argk, K), axis=1, keepdims=True)
        idx_ref[0, t:t + TT, :] = idx
        losspart += jnp.sum(mind).reshape(1, 1)
    mind = losspart

    @pl.when(b == 0)
    def _():
        loss_ref[...] = jnp.zeros((1, 1), jnp.float32)

    loss_ref[...] += jnp.sum(mind).reshape(1, 1)


_TB = 2304


def _encode(zf, codebook):
    N, D = zf.shape
    K = codebook.shape[0]
    G = N // _TB
    idx3, losssum = pl.pallas_call(
        _encode_body,
        grid=(G,),
        in_specs=[
            pl.BlockSpec((_TB, D), lambda i: (i, 0)),
            pl.BlockSpec((K, D), lambda i: (0, 0)),
        ],
        out_specs=[
            pl.BlockSpec((1, _TB, 1), lambda i: (i, 0, 0)),
            pl.BlockSpec((1, 1), lambda i: (0, 0)),
        ],
        out_shape=[
            jax.ShapeDtypeStruct((G, _TB, 1), jnp.int32),
            jax.ShapeDtypeStruct((1, 1), jnp.float32),
        ],
        scratch_shapes=[pltpu.VMEM((1, K), jnp.float32)],
    )(zf, codebook)
    return idx3.reshape(N), losssum[0, 0]


_NC, _NS = 2, 16
_NW = _NC * _NS
_CHUNK = 96


_NB = 4
_INFLIGHT = 3


def _sc_gather_body(per, cb_hbm, idx_hbm, out_hbm, idx_v, buf0, buf1, buf2,
                    buf3, gs0, gs1, gs2, gs3, os0, os1, os2, os3):
    wid = lax.axis_index("s") * _NC + lax.axis_index("c")
    wbase = wid * per
    pltpu.sync_copy(idx_hbm.at[pl.ds(wbase, per)], idx_v)
    bufs = (buf0, buf1, buf2, buf3)
    gsems = (gs0, gs1, gs2, gs3)
    osems = (os0, os1, os2, os3)
    nch = per // _CHUNK
    g = [None] * nch
    o = [None] * nch
    o_waited = [False] * nch

    def _gather(j):
        return pltpu.async_copy(
            cb_hbm.at[idx_v.at[pl.ds(j * _CHUNK, _CHUNK)]],
            bufs[j % _NB], gsems[j % _NB])

    for j in range(min(_INFLIGHT, nch)):
        g[j] = _gather(j)
    for j in range(nch):
        g[j].wait()
        o[j] = pltpu.async_copy(
            bufs[j % _NB],
            out_hbm.at[pl.ds(wbase + j * _CHUNK, _CHUNK)],
            osems[j % _NB])
        nj = j + _INFLIGHT
        if nj < nch:
            prev = nj - _NB
            if prev >= 0 and not o_waited[prev]:
                o[prev].wait()
                o_waited[prev] = True
            g[nj] = _gather(nj)
    for j in range(nch):
        if not o_waited[j]:
            o[j].wait()


def _sc_gather(codebook, idx_flat):
    n = idx_flat.shape[0]
    K, D = codebook.shape
    per = n // _NW
    mesh = plsc.VectorSubcoreMesh(core_axis_name="c", subcore_axis_name="s")
    fn = functools.partial(
        pl.kernel,
        out_type=jax.ShapeDtypeStruct((n, D), jnp.float32),
        mesh=mesh,
        scratch_types=(
            [pltpu.VMEM((per,), jnp.int32)]
            + [pltpu.VMEM((_CHUNK, D), jnp.float32) for _ in range(_NB)]
            + [pltpu.SemaphoreType.DMA for _ in range(2 * _NB)]
        ),
    )(functools.partial(_sc_gather_body, per))
    return fn(codebook, idx_flat)


def kernel(z, codebook):
    B, T, D = z.shape
    idx_flat, losssum = _encode(z.reshape(-1, D), codebook)
    loss = losssum * ((1.0 + COMMIT) / (B * T * D))
    quant = _sc_gather(codebook, idx_flat)
    return quant.reshape(B, T, D), loss, idx_flat.reshape(B, T)


# speedup vs baseline: 1.3275x; 1.0132x over previous
import functools

import jax
import jax.numpy as jnp
from jax import lax
from jax.experimental import pallas as pl
from jax.experimental.pallas import tpu as pltpu
from jax.experimental.pallas import tpu_sc as plsc

COMMIT = 0.25


def _encode_body(z_ref, cb_ref, idx_ref, loss_ref, cnorm_ref):
    b = pl.program_id(0)
    zb = z_ref[...]
    cb = cb_ref[...]
    T, D = zb.shape
    K = cb.shape[0]
    LW = 128

    @pl.when(b == 0)
    def _():
        cnorm_ref[...] = jnp.sum(cb * cb, axis=1)[None, :]

    mm2 = lax.dot_general(zb + zb, cb, (((1,), (1,)), ((), ())),
                          preferred_element_type=jnp.float32)
    znorm = jnp.sum(zb * zb, axis=1, keepdims=True)
    TT = 64
    lane = lax.broadcasted_iota(jnp.int32, (TT, LW), 1)
    losspart = jnp.zeros((1, 1), jnp.float32)
    for t in range(0, T, TT):
        zn = znorm[t:t + TT, :]
        m = argk = None
        for j in range(K // LW):
            cn = cnorm_ref[:, j * LW:(j + 1) * LW]
            dj = zn - mm2[t:t + TT, j * LW:(j + 1) * LW] + cn
            if j == 0:
                m, argk = dj, lane
            else:
                lt = dj < m
                m = jnp.where(lt, dj, m)
                argk = jnp.where(lt, lane + j * LW, argk)
        mind = jnp.min(m, axis=1, keepdims=True)
        idx = jnp.min(jnp.where(m == mind, argk, K), axis=1, keepdims=True)
        idx_ref[0, t:t + TT, :] = idx
        losspart += jnp.sum(mind).reshape(1, 1)
    mind = losspart

    @pl.when(b == 0)
    def _():
        loss_ref[...] = jnp.zeros((1, 1), jnp.float32)

    loss_ref[...] += jnp.sum(mind).reshape(1, 1)


_TB = 2304


def _encode(zf, codebook):
    N, D = zf.shape
    K = codebook.shape[0]
    G = N // _TB
    idx3, losssum = pl.pallas_call(
        _encode_body,
        grid=(G,),
        in_specs=[
            pl.BlockSpec((_TB, D), lambda i: (i, 0)),
            pl.BlockSpec((K, D), lambda i: (0, 0)),
        ],
        out_specs=[
            pl.BlockSpec((1, _TB, 1), lambda i: (i, 0, 0)),
            pl.BlockSpec((1, 1), lambda i: (0, 0)),
        ],
        out_shape=[
            jax.ShapeDtypeStruct((G, _TB, 1), jnp.int32),
            jax.ShapeDtypeStruct((1, 1), jnp.float32),
        ],
        scratch_shapes=[pltpu.VMEM((1, K), jnp.float32)],
    )(zf, codebook)
    return idx3.reshape(N), losssum[0, 0]


_NC, _NS = 2, 16
_NW = _NC * _NS
_CHUNK = 96


_NB = 4
_INFLIGHT = 3


def _sc_gather_body(per, cb_hbm, idx_hbm, out_hbm, idx_v, buf0, buf1, buf2,
                    buf3, gs0, gs1, gs2, gs3, os0, os1, os2, os3):
    wid = lax.axis_index("s") * _NC + lax.axis_index("c")
    wbase = wid * per
    pltpu.sync_copy(idx_hbm.at[pl.ds(wbase, per)], idx_v)
    bufs = (buf0, buf1, buf2, buf3)
    gsems = (gs0, gs1, gs2, gs3)
    osems = (os0, os1, os2, os3)
    nch = per // _CHUNK
    g = [None] * nch
    o = [None] * nch
    o_waited = [False] * nch

    def _gather(j):
        return pltpu.async_copy(
            cb_hbm.at[idx_v.at[pl.ds(j * _CHUNK, _CHUNK)]],
            bufs[j % _NB], gsems[j % _NB])

    for j in range(min(_INFLIGHT, nch)):
        g[j] = _gather(j)
    for j in range(nch):
        g[j].wait()
        o[j] = pltpu.async_copy(
            bufs[j % _NB],
            out_hbm.at[pl.ds(wbase + j * _CHUNK, _CHUNK)],
            osems[j % _NB])
        nj = j + _INFLIGHT
        if nj < nch:
            prev = nj - _NB
            if prev >= 0 and not o_waited[prev]:
                o[prev].wait()
                o_waited[prev] = True
            g[nj] = _gather(nj)
    for j in range(nch):
        if not o_waited[j]:
            o[j].wait()


def _sc_gather(codebook, idx_flat):
    n = idx_flat.shape[0]
    K, D = codebook.shape
    per = n // _NW
    mesh = plsc.VectorSubcoreMesh(core_axis_name="c", subcore_axis_name="s")
    fn = functools.partial(
        pl.kernel,
        out_type=jax.ShapeDtypeStruct((n, D), jnp.float32),
        mesh=mesh,
        scratch_types=(
            [pltpu.VMEM((per,), jnp.int32)]
            + [pltpu.VMEM((_CHUNK, D), jnp.float32) for _ in range(_NB)]
            + [pltpu.SemaphoreType.DMA for _ in range(2 * _NB)]
        ),
    )(functools.partial(_sc_gather_body, per))
    return fn(codebook, idx_flat)


def kernel(z, codebook):
    B, T, D = z.shape
    idx_flat, losssum = _encode(z.reshape(-1, D), codebook)
    loss = losssum * ((1.0 + COMMIT) / (B * T * D))
    quant = _sc_gather(codebook, idx_flat)
    return quant.reshape(B, T, D), loss, idx_flat.reshape(B, T)
